# Initial kernel scaffold; baseline (speedup 1.0000x reference)
import jax
import jax.numpy as jnp
from jax import lax
from jax.experimental import pallas as pl
from jax.experimental.pallas import tpu as pltpu
from jax.experimental.pallas import tpu_sc as plsc

_N = 10000
_D = 128
_E = 320000
_NC = 2
_NS = 16
_NW = _NC * _NS
_K = 128
_CPW = 80
_EPW = _CPW * _K
_EPAD = _NW * _EPW
_EPT = _EPAD // _NS
_NPAD = 10240
_RPT = _NPAD // _NS
_RWB = _N // _NS
_WB_CHUNKS = ((0, 128), (128, 128), (256, 128), (384, 128), (512, 113))


def _deg_body(dst_hbm, ew_hbm, dis_hbm, dst_v, ew_v, part_v, shared, acc_v, tmp_v):
    cid = lax.axis_index("c")
    sid = lax.axis_index("s")

    @pl.when(cid == 0)
    def _():
        pltpu.sync_copy(dst_hbm.at[sid], dst_v)
        pltpu.sync_copy(ew_hbm.at[sid], ew_v)
        zeros = jnp.zeros((16,), jnp.float32)

        def zero_body(i, _):
            part_v[pl.ds(i * 16, 16)] = zeros
            return 0

        lax.fori_loop(0, _NPAD // 16, zero_body, 0)

        def scat_body(g, _):
            idx = dst_v[pl.ds(g * 16, 16)]
            w = ew_v[pl.ds(g * 16, 16)]
            plsc.addupdate_scatter(part_v, [idx], w)
            return 0

        lax.fori_loop(0, _EPT // 16, scat_body, 0)

        pltpu.sync_copy(part_v, shared.at[sid])
        plsc.subcore_barrier()

        base = sid * _RPT
        pltpu.sync_copy(shared.at[0, pl.ds(base, _RPT)], acc_v)
        for k in range(1, _NS):
            pltpu.sync_copy(shared.at[k, pl.ds(base, _RPT)], tmp_v)

            def add_body(i, _):
                sl = pl.ds(i * 16, 16)
                acc_v[sl] = acc_v[sl] + tmp_v[sl]
                return 0

            lax.fori_loop(0, _RPT // 16, add_body, 0)

        def newton_body(i, _):
            sl = pl.ds(i * 16, 16)
            xv = acc_v[sl] + 1.0
            bits = plsc.bitcast(xv, jnp.int32)
            y = plsc.bitcast(
                jnp.int32(0x5F3759DF) - lax.shift_right_logical(bits, 1),
                jnp.float32,
            )
            half = xv * 0.5
            for _u in range(3):
                y = y * (1.5 - half * y * y)
            acc_v[sl] = y
            return 0

        lax.fori_loop(0, _RPT // 16, newton_body, 0)
        pltpu.sync_copy(acc_v, dis_hbm.at[pl.ds(base, _RPT)])


_deg_kernel = pl.kernel(
    _deg_body,
    out_type=jax.ShapeDtypeStruct((_NPAD,), jnp.float32),
    mesh=plsc.VectorSubcoreMesh(core_axis_name="c", subcore_axis_name="s"),
    scratch_types=[
        pltpu.VMEM((_EPT,), jnp.int32),
        pltpu.VMEM((_EPT,), jnp.float32),
        pltpu.VMEM((_NPAD,), jnp.float32),
        pltpu.VMEM_SHARED((_NS, _NPAD), jnp.float32),
        pltpu.VMEM((_RPT,), jnp.float32),
        pltpu.VMEM((_RPT,), jnp.float32),
    ],
)


def _msg_body(g_hbm, src_hbm, dst_hbm, ew_hbm, out_hbm,
              src_v, dst_v, ew_v, buf, acc, sem):
    cid = lax.axis_index("c")
    sid = lax.axis_index("s")
    wid = sid * _NC + cid

    pltpu.sync_copy(src_hbm.at[wid], src_v)
    pltpu.sync_copy(dst_hbm.at[wid], dst_v)
    pltpu.sync_copy(ew_hbm.at[wid], ew_v)

    zeros = jnp.zeros((16,), jnp.float32)

    def zb(r, _):
        for cc in range(8):
            buf[0, r, pl.ds(cc * 16, 16)] = zeros
        return 0

    lax.fori_loop(0, _K, zb, 0)

    zbase = sid * _RWB
    for off, n in _WB_CHUNKS:
        pltpu.sync_copy(buf.at[0, pl.ds(0, n)], acc.at[pl.ds(zbase + off, n)])
    plsc.subcore_barrier()

    def chunk_body(c, _):
        pltpu.async_copy(g_hbm.at[src_v.at[c]], buf.at[0], sem).wait()
        cvec = jnp.full((16,), c, jnp.int32)
        for r in range(_K):
            ewb = plsc.load_gather(ew_v, [cvec, jnp.full((16,), r, jnp.int32)])
            for cc in range(8):
                sl = pl.ds(cc * 16, 16)
                buf[0, r, sl] = buf[0, r, sl] * ewb
        pltpu.sync_copy(buf.at[0], acc.at[dst_v.at[c]], add=True)
        return 0

    lax.fori_loop(0, _CPW, chunk_body, 0)
    plsc.subcore_barrier()

    for off, n in _WB_CHUNKS:
        pltpu.sync_copy(acc.at[pl.ds(zbase + off, n)],
                        out_hbm.at[cid, pl.ds(zbase + off, n)])


_msg_kernel = pl.kernel(
    _msg_body,
    out_type=jax.ShapeDtypeStruct((_NC, _N, _D), jnp.float32),
    mesh=plsc.VectorSubcoreMesh(core_axis_name="c", subcore_axis_name="s"),
    scratch_types=[
        pltpu.VMEM((_CPW, _K), jnp.int32),
        pltpu.VMEM((_CPW, _K), jnp.int32),
        pltpu.VMEM((_CPW, _K), jnp.float32),
        pltpu.VMEM((2, _K, _D), jnp.float32),
        pltpu.VMEM_SHARED((_N, _D), jnp.float32),
        pltpu.SemaphoreType.DMA,
    ],
)


def _tc1_body(x_ref, w_ref, dis_ref, g_ref):
    h = jnp.dot(x_ref[...], w_ref[...], preferred_element_type=jnp.float32)
    g_ref[...] = h * dis_ref[...]


def _tc2_body(p_ref, g_ref, dis_ref, b_ref, w_ref, out_ref):
    s = (p_ref[0] + p_ref[1] + g_ref[...]) * dis_ref[...] + b_ref[...]
    t = jnp.maximum(s, 0.0)
    out_ref[...] = jnp.dot(t, w_ref[...],
                           preferred_element_type=jnp.float32) * dis_ref[...]


def _tc3_body(p_ref, g_ref, dis_ref, b_ref, out_ref):
    out_ref[...] = (p_ref[0] + p_ref[1] + g_ref[...]) * dis_ref[...] + b_ref[...]


def kernel(x, edge_index, edge_weight, W1, b1, W2, b2):
    src = edge_index[0].astype(jnp.int32)
    dst = edge_index[1].astype(jnp.int32)
    ew = edge_weight.astype(jnp.float32)

    pad = _EPAD - _E
    src_p = jnp.concatenate([src, jnp.zeros((pad,), jnp.int32)])
    dst_p = jnp.concatenate([dst, jnp.zeros((pad,), jnp.int32)])
    ew_p = jnp.concatenate([ew, jnp.zeros((pad,), jnp.float32)])
    src3 = src_p.reshape(_NW, _CPW, _K)
    dst3 = dst_p.reshape(_NW, _CPW, _K)
    ew3 = ew_p.reshape(_NW, _CPW, _K)
    dst2 = dst_p.reshape(_NS, _EPT)
    ew2 = ew_p.reshape(_NS, _EPT)

    dis_full = _deg_kernel(dst2, ew2)
    dis_col = dis_full[:_N].reshape(_N, 1)

    g1 = pl.pallas_call(
        _tc1_body,
        out_shape=jax.ShapeDtypeStruct((_N, _D), jnp.float32),
    )(x, W1, dis_col)

    p1 = _msg_kernel(g1, src3, dst3, ew3)

    g2 = pl.pallas_call(
        _tc2_body,
        out_shape=jax.ShapeDtypeStruct((_N, _D), jnp.float32),
    )(p1, g1, dis_col, b1.reshape(1, _D), W2)

    p2 = _msg_kernel(g2, src3, dst3, ew3)

    out = pl.pallas_call(
        _tc3_body,
        out_shape=jax.ShapeDtypeStruct((_N, _D), jnp.float32),
    )(p2, g2, dis_col, b2.reshape(1, _D))

    return out


# speedup vs baseline: 8.2376x; 8.2376x over previous
import jax
import jax.numpy as jnp
from jax import lax
from jax.experimental import pallas as pl
from jax.experimental.pallas import tpu as pltpu
from jax.experimental.pallas import tpu_sc as plsc

_N = 10000
_D = 128
_DH = _D // 2
_E = 320000
_NC = 2
_NS = 16
_NW = _NC * _NS
_K = 128
_CPT = 160
_EPT = _CPT * _K
_EPAD = _NS * _EPT
_NPAD = 10240
_RPT = _NPAD // _NS
_WB_CHUNKS = ((0, 128), (128, 128), (256, 128), (384, 128), (512, 128))


def _deg_body(dst_hbm, ew_hbm, dis_hbm, dst_v, ew_v, part_v, shared, acc_v, tmp_v):
    cid = lax.axis_index("c")
    sid = lax.axis_index("s")

    @pl.when(cid == 0)
    def _():
        pltpu.sync_copy(dst_hbm.at[sid, 0], dst_v)
        pltpu.sync_copy(ew_hbm.at[sid, 0], ew_v)
        zeros = jnp.zeros((16,), jnp.float32)

        def zero_body(i, _):
            part_v[pl.ds(i * 16, 16)] = zeros
            return 0

        lax.fori_loop(0, _NPAD // 16, zero_body, 0)

        def scat_body(g, _):
            idx = dst_v[pl.ds(g * 16, 16)]
            w = ew_v[pl.ds(g * 16, 16)]
            plsc.addupdate_scatter(part_v, [idx], w)
            return 0

        lax.fori_loop(0, _EPT // 16, scat_body, 0)

        pltpu.sync_copy(part_v, shared.at[sid])
        plsc.subcore_barrier()

        base = sid * _RPT
        pltpu.sync_copy(shared.at[0, pl.ds(base, _RPT)], acc_v)
        for k in range(1, _NS):
            pltpu.sync_copy(shared.at[k, pl.ds(base, _RPT)], tmp_v)

            def add_body(i, _):
                sl = pl.ds(i * 16, 16)
                acc_v[sl] = acc_v[sl] + tmp_v[sl]
                return 0

            lax.fori_loop(0, _RPT // 16, add_body, 0)

        def newton_body(i, _):
            sl = pl.ds(i * 16, 16)
            xv = acc_v[sl] + 1.0
            bits = plsc.bitcast(xv, jnp.int32)
            y = plsc.bitcast(
                jnp.int32(0x5F3759DF) - lax.shift_right_logical(bits, 1),
                jnp.float32,
            )
            half = xv * 0.5
            for _u in range(3):
                y = y * (1.5 - half * y * y)
            acc_v[sl] = y
            return 0

        lax.fori_loop(0, _RPT // 16, newton_body, 0)
        pltpu.sync_copy(acc_v, dis_hbm.at[pl.ds(base, _RPT)])


_deg_kernel = pl.kernel(
    _deg_body,
    out_type=jax.ShapeDtypeStruct((_NPAD,), jnp.float32),
    mesh=plsc.VectorSubcoreMesh(core_axis_name="c", subcore_axis_name="s"),
    scratch_types=[
        pltpu.VMEM((_EPT,), jnp.int32),
        pltpu.VMEM((_EPT,), jnp.float32),
        pltpu.VMEM((_NPAD,), jnp.float32),
        pltpu.VMEM_SHARED((_NS, _NPAD), jnp.float32),
        pltpu.VMEM((_RPT,), jnp.float32),
        pltpu.VMEM((_RPT,), jnp.float32),
    ],
    compiler_params=pltpu.CompilerParams(needs_layout_passes=False),
)


def _msg_body(g_hbm, src_hbm, dst_hbm, ew_hbm, out_hbm,
              src_v, dst_v, ew_v, buf, acc, sem):
    cid = lax.axis_index("c")
    sid = lax.axis_index("s")

    pltpu.sync_copy(src_hbm.at[sid], src_v)
    pltpu.sync_copy(dst_hbm.at[sid], dst_v)
    pltpu.sync_copy(ew_hbm.at[sid], ew_v)

    zeros = jnp.zeros((16,), jnp.float32)

    def zb(r, _):
        for cc in range(_DH // 16):
            buf[0, r, pl.ds(cc * 16, 16)] = zeros
        return 0

    lax.fori_loop(0, _K, zb, 0)

    zbase = sid * _RPT
    for off, n in _WB_CHUNKS:
        pltpu.sync_copy(buf.at[0, pl.ds(0, n)], acc.at[pl.ds(zbase + off, n)])
    plsc.subcore_barrier()

    def chunk_body(c, _):
        pltpu.async_copy(g_hbm.at[cid].at[src_v.at[c]], buf.at[0], sem).wait()
        cvec = jnp.full((16,), c, jnp.int32)
        for r in range(_K):
            ewb = plsc.load_gather(ew_v, [cvec, jnp.full((16,), r, jnp.int32)])
            for cc in range(_DH // 16):
                sl = pl.ds(cc * 16, 16)
                buf[0, r, sl] = buf[0, r, sl] * ewb
        pltpu.sync_copy(buf.at[0], acc.at[dst_v.at[c]], add=True)
        return 0

    lax.fori_loop(0, _CPT, chunk_body, 0)
    plsc.subcore_barrier()

    for off, n in _WB_CHUNKS:
        pltpu.sync_copy(acc.at[pl.ds(zbase + off, n)],
                        out_hbm.at[cid, pl.ds(zbase + off, n)])


_msg_kernel = pl.kernel(
    _msg_body,
    out_type=jax.ShapeDtypeStruct((_NC, _NPAD, _DH), jnp.float32),
    mesh=plsc.VectorSubcoreMesh(core_axis_name="c", subcore_axis_name="s"),
    scratch_types=[
        pltpu.VMEM((_CPT, _K), jnp.int32),
        pltpu.VMEM((_CPT, _K), jnp.int32),
        pltpu.VMEM((_CPT, _K), jnp.float32),
        pltpu.VMEM((2, _K, _DH), jnp.float32),
        pltpu.VMEM_SHARED((_NPAD, _DH), jnp.float32),
        pltpu.SemaphoreType.DMA,
    ],
    compiler_params=pltpu.CompilerParams(needs_layout_passes=False,
                                         use_tc_tiling_on_sc=False),
)


def _tc1_body(x_ref, w_ref, dis_ref, g_ref):
    h = jnp.dot(x_ref[...], w_ref[...], preferred_element_type=jnp.float32)
    g = h * dis_ref[...]
    g_ref[0] = g[:, :_DH]
    g_ref[1] = g[:, _DH:]


def _tc2_body(p_ref, g_ref, dis_ref, b_ref, w_ref, out_ref):
    p = jnp.concatenate([p_ref[0, pl.ds(0, _N), :],
                         p_ref[1, pl.ds(0, _N), :]], axis=-1)
    g = jnp.concatenate([g_ref[0], g_ref[1]], axis=-1)
    s = (p + g) * dis_ref[...] + b_ref[...]
    t = jnp.maximum(s, 0.0)
    h = jnp.dot(t, w_ref[...], preferred_element_type=jnp.float32)
    g2 = h * dis_ref[...]
    out_ref[0] = g2[:, :_DH]
    out_ref[1] = g2[:, _DH:]


def _tc3_body(p_ref, g_ref, dis_ref, b_ref, out_ref):
    p = jnp.concatenate([p_ref[0, pl.ds(0, _N), :],
                         p_ref[1, pl.ds(0, _N), :]], axis=-1)
    g = jnp.concatenate([g_ref[0], g_ref[1]], axis=-1)
    out_ref[...] = (p + g) * dis_ref[...] + b_ref[...]


def kernel(x, edge_index, edge_weight, W1, b1, W2, b2):
    src = edge_index[0].astype(jnp.int32)
    dst = edge_index[1].astype(jnp.int32)
    ew = edge_weight.astype(jnp.float32)

    pad = _EPAD - _E
    src_p = jnp.concatenate([src, jnp.zeros((pad,), jnp.int32)])
    dst_p = jnp.concatenate([dst, jnp.zeros((pad,), jnp.int32)])
    ew_p = jnp.concatenate([ew, jnp.zeros((pad,), jnp.float32)])
    src3 = src_p.reshape(_NS, _CPT, _K)
    dst3 = dst_p.reshape(_NS, _CPT, _K)
    ew3 = ew_p.reshape(_NS, _CPT, _K)
    dst2 = dst_p.reshape(_NS, 1, _EPT)
    ew2 = ew_p.reshape(_NS, 1, _EPT)

    dis_full = _deg_kernel(dst2, ew2)
    dis_col = dis_full[:_N].reshape(_N, 1)

    g1 = pl.pallas_call(
        _tc1_body,
        out_shape=jax.ShapeDtypeStruct((_NC, _N, _DH), jnp.float32),
    )(x, W1, dis_col)

    p1 = _msg_kernel(g1, src3, dst3, ew3)

    g2 = pl.pallas_call(
        _tc2_body,
        out_shape=jax.ShapeDtypeStruct((_NC, _N, _DH), jnp.float32),
    )(p1, g1, dis_col, b1.reshape(1, _D), W2)

    p2 = _msg_kernel(g2, src3, dst3, ew3)

    out = pl.pallas_call(
        _tc3_body,
        out_shape=jax.ShapeDtypeStruct((_N, _D), jnp.float32),
    )(p2, g2, dis_col, b2.reshape(1, _D))

    return out


# speedup vs baseline: 10.0366x; 1.2184x over previous
import jax
import jax.numpy as jnp
from jax import lax
from jax.experimental import pallas as pl
from jax.experimental.pallas import tpu as pltpu
from jax.experimental.pallas import tpu_sc as plsc

_N = 10000
_D = 128
_DH = _D // 2
_E = 320000
_NC = 2
_NS = 16
_NW = _NC * _NS
_K = 128
_CPT = 160
_EPT = _CPT * _K
_EPAD = _NS * _EPT
_NPAD = 10240
_RPT = _NPAD // _NS
_WB_CHUNKS = ((0, 128), (128, 128), (256, 128), (384, 128), (512, 128))


def _deg_body(dst_hbm, ew_hbm, dis_hbm, dst_v, ew_v, part_v, shared, acc_v, tmp_v):
    cid = lax.axis_index("c")
    sid = lax.axis_index("s")

    @pl.when(cid == 0)
    def _():
        pltpu.sync_copy(dst_hbm.at[sid, 0], dst_v)
        pltpu.sync_copy(ew_hbm.at[sid, 0], ew_v)
        zeros = jnp.zeros((16,), jnp.float32)

        def zero_body(i, _):
            part_v[pl.ds(i * 16, 16)] = zeros
            return 0

        lax.fori_loop(0, _NPAD // 16, zero_body, 0)

        def scat_body(g, _):
            idx = dst_v[pl.ds(g * 16, 16)]
            w = ew_v[pl.ds(g * 16, 16)]
            plsc.addupdate_scatter(part_v, [idx], w)
            return 0

        lax.fori_loop(0, _EPT // 16, scat_body, 0)

        pltpu.sync_copy(part_v, shared.at[sid])
        plsc.subcore_barrier()

        base = sid * _RPT
        pltpu.sync_copy(shared.at[0, pl.ds(base, _RPT)], acc_v)
        for k in range(1, _NS):
            pltpu.sync_copy(shared.at[k, pl.ds(base, _RPT)], tmp_v)

            def add_body(i, _):
                sl = pl.ds(i * 16, 16)
                acc_v[sl] = acc_v[sl] + tmp_v[sl]
                return 0

            lax.fori_loop(0, _RPT // 16, add_body, 0)

        def newton_body(i, _):
            sl = pl.ds(i * 16, 16)
            xv = acc_v[sl] + 1.0
            bits = plsc.bitcast(xv, jnp.int32)
            y = plsc.bitcast(
                jnp.int32(0x5F3759DF) - lax.shift_right_logical(bits, 1),
                jnp.float32,
            )
            half = xv * 0.5
            for _u in range(3):
                y = y * (1.5 - half * y * y)
            acc_v[sl] = y
            return 0

        lax.fori_loop(0, _RPT // 16, newton_body, 0)
        pltpu.sync_copy(acc_v, dis_hbm.at[pl.ds(base, _RPT)])


_deg_kernel = pl.kernel(
    _deg_body,
    out_type=jax.ShapeDtypeStruct((_NPAD,), jnp.float32),
    mesh=plsc.VectorSubcoreMesh(core_axis_name="c", subcore_axis_name="s"),
    scratch_types=[
        pltpu.VMEM((_EPT,), jnp.int32),
        pltpu.VMEM((_EPT,), jnp.float32),
        pltpu.VMEM((_NPAD,), jnp.float32),
        pltpu.VMEM_SHARED((_NS, _NPAD), jnp.float32),
        pltpu.VMEM((_RPT,), jnp.float32),
        pltpu.VMEM((_RPT,), jnp.float32),
    ],
    compiler_params=pltpu.CompilerParams(needs_layout_passes=False),
)


def _msg_body(g_hbm, src_hbm, dst_hbm, ew_hbm, out_hbm,
              src_v, dst_v, ew_v, buf, acc, gs0, gs1, ss0, ss1):
    cid = lax.axis_index("c")
    sid = lax.axis_index("s")
    gsem = (gs0, gs1)
    ssem = (ss0, ss1)

    pltpu.sync_copy(src_hbm.at[sid], src_v)
    pltpu.sync_copy(dst_hbm.at[sid], dst_v)
    pltpu.sync_copy(ew_hbm.at[sid], ew_v)

    zeros = jnp.zeros((16,), jnp.float32)

    def zb(r, _):
        for cc in range(_DH // 16):
            buf[0, r, pl.ds(cc * 16, 16)] = zeros
        return 0

    lax.fori_loop(0, _K, zb, 0)

    zbase = sid * _RPT
    for off, n in _WB_CHUNKS:
        pltpu.sync_copy(buf.at[0, pl.ds(0, n)], acc.at[pl.ds(zbase + off, n)])
    plsc.subcore_barrier()

    def _gather(c, b, sem):
        return pltpu.async_copy(g_hbm.at[cid].at[src_v.at[c]], buf.at[b], sem)

    _gather(0, 0, gsem[0])
    _gather(1, 1, gsem[1])

    def chunk_body(i, _):
        for b in range(2):
            c = 2 * i + b
            pltpu.make_async_copy(g_hbm.at[cid].at[src_v.at[c]],
                                  buf.at[b], gsem[b]).wait()
            cvec = jnp.full((16,), c, jnp.int32)
            for r in range(_K):
                ewb = plsc.load_gather(
                    ew_v, [cvec, jnp.full((16,), r, jnp.int32)])
                for cc in range(_DH // 16):
                    sl = pl.ds(cc * 16, 16)
                    buf[b, r, sl] = buf[b, r, sl] * ewb
            pltpu.async_copy(buf.at[b], acc.at[dst_v.at[c]], ssem[b], add=True)

            @pl.when(i < _CPT // 2 - 1)
            def _():
                pltpu.make_async_copy(buf.at[b], acc.at[dst_v.at[c]],
                                      ssem[b]).wait()
                _gather(c + 2, b, gsem[b])

        return 0

    lax.fori_loop(0, _CPT // 2, chunk_body, 0)
    for b in range(2):
        c = _CPT - 2 + b
        pltpu.make_async_copy(buf.at[b], acc.at[dst_v.at[c]], ssem[b]).wait()
    plsc.subcore_barrier()

    for off, n in _WB_CHUNKS:
        pltpu.sync_copy(acc.at[pl.ds(zbase + off, n)],
                        out_hbm.at[cid, pl.ds(zbase + off, n)])


_msg_kernel = pl.kernel(
    _msg_body,
    out_type=jax.ShapeDtypeStruct((_NC, _NPAD, _DH), jnp.float32),
    mesh=plsc.VectorSubcoreMesh(core_axis_name="c", subcore_axis_name="s"),
    scratch_types=[
        pltpu.VMEM((_CPT, _K), jnp.int32),
        pltpu.VMEM((_CPT, _K), jnp.int32),
        pltpu.VMEM((_CPT, _K), jnp.float32),
        pltpu.VMEM((2, _K, _DH), jnp.float32),
        pltpu.VMEM_SHARED((_NPAD, _DH), jnp.float32),
        pltpu.SemaphoreType.DMA,
        pltpu.SemaphoreType.DMA,
        pltpu.SemaphoreType.DMA,
        pltpu.SemaphoreType.DMA,
    ],
    compiler_params=pltpu.CompilerParams(needs_layout_passes=False,
                                         use_tc_tiling_on_sc=False),
)


def _tc1_body(x_ref, w_ref, dis_ref, g_ref):
    h = jnp.dot(x_ref[...], w_ref[...], preferred_element_type=jnp.float32)
    g = h * dis_ref[...]
    g_ref[0] = g[:, :_DH]
    g_ref[1] = g[:, _DH:]


def _tc2_body(p_ref, g_ref, dis_ref, b_ref, w_ref, out_ref):
    p = jnp.concatenate([p_ref[0, pl.ds(0, _N), :],
                         p_ref[1, pl.ds(0, _N), :]], axis=-1)
    g = jnp.concatenate([g_ref[0], g_ref[1]], axis=-1)
    s = (p + g) * dis_ref[...] + b_ref[...]
    t = jnp.maximum(s, 0.0)
    h = jnp.dot(t, w_ref[...], preferred_element_type=jnp.float32)
    g2 = h * dis_ref[...]
    out_ref[0] = g2[:, :_DH]
    out_ref[1] = g2[:, _DH:]


def _tc3_body(p_ref, g_ref, dis_ref, b_ref, out_ref):
    p = jnp.concatenate([p_ref[0, pl.ds(0, _N), :],
                         p_ref[1, pl.ds(0, _N), :]], axis=-1)
    g = jnp.concatenate([g_ref[0], g_ref[1]], axis=-1)
    out_ref[...] = (p + g) * dis_ref[...] + b_ref[...]


def kernel(x, edge_index, edge_weight, W1, b1, W2, b2):
    src = edge_index[0].astype(jnp.int32)
    dst = edge_index[1].astype(jnp.int32)
    ew = edge_weight.astype(jnp.float32)

    pad = _EPAD - _E
    src_p = jnp.concatenate([src, jnp.zeros((pad,), jnp.int32)])
    dst_p = jnp.concatenate([dst, jnp.zeros((pad,), jnp.int32)])
    ew_p = jnp.concatenate([ew, jnp.zeros((pad,), jnp.float32)])
    src3 = src_p.reshape(_NS, _CPT, _K)
    dst3 = dst_p.reshape(_NS, _CPT, _K)
    ew3 = ew_p.reshape(_NS, _CPT, _K)
    dst2 = dst_p.reshape(_NS, 1, _EPT)
    ew2 = ew_p.reshape(_NS, 1, _EPT)

    dis_full = _deg_kernel(dst2, ew2)
    dis_col = dis_full[:_N].reshape(_N, 1)

    g1 = pl.pallas_call(
        _tc1_body,
        out_shape=jax.ShapeDtypeStruct((_NC, _N, _DH), jnp.float32),
    )(x, W1, dis_col)

    p1 = _msg_kernel(g1, src3, dst3, ew3)

    g2 = pl.pallas_call(
        _tc2_body,
        out_shape=jax.ShapeDtypeStruct((_NC, _N, _DH), jnp.float32),
    )(p1, g1, dis_col, b1.reshape(1, _D), W2)

    p2 = _msg_kernel(g2, src3, dst3, ew3)

    out = pl.pallas_call(
        _tc3_body,
        out_shape=jax.ShapeDtypeStruct((_N, _D), jnp.float32),
    )(p2, g2, dis_col, b2.reshape(1, _D))

    return out
